# baseline (speedup 1.0000x reference)
import dataclasses
import functools

import jax
import jax.numpy as jnp
from jax.experimental import pallas as pl
from jax.experimental.pallas import tpu as pltpu
from jax.experimental.pallas import tpu_sc as plsc

V, D, H = 1000000, 32, 64
B, L = 4096, 200
N = B * L
PACK = 4
NP = N // PACK
DP, HP = D * PACK, H * PACK

GATHER_WINDOW = 128
MLP_BLOCK = 1024


def _sc_gather(table, x_chunk):
    mesh = plsc.VectorSubcoreMesh(core_axis_name="c", subcore_axis_name="s")
    BC = x_chunk.shape[0]

    n_tiles = 32
    rows_per_tile = BC // n_tiles
    XR = 4
    n_steps = rows_per_tile // XR
    assert n_steps % 2 == 0
    TOK = XR * L
    PR = TOK // PACK
    NPC = BC * L // PACK
    CHUNKS = ((0, 104), (104, 96))

    @functools.partial(
        pl.kernel,
        out_type=jax.ShapeDtypeStruct((NPC, DP), jnp.float32),
        mesh=mesh,
        scratch_types=[
            pltpu.VMEM((2, XR, L), jnp.int32),
            pltpu.VMEM((2, TOK, D), jnp.float32),
            pltpu.VMEM((2, PR, DP), jnp.float32),
            pltpu.SemaphoreType.DMA, pltpu.SemaphoreType.DMA,
            pltpu.SemaphoreType.DMA, pltpu.SemaphoreType.DMA,
            pltpu.SemaphoreType.DMA, pltpu.SemaphoreType.DMA,
        ],
        compiler_params=pltpu.CompilerParams(use_tc_tiling_on_sc=False),
    )
    def gather_kernel(table_hbm, idx_hbm, out_hbm, idx_v, rows_v,
                      packed_v, si0, si1, sg0, sg1, so0, so1):
        from jax import lax
        wid = lax.axis_index("s") * 2 + lax.axis_index("c")
        base = wid * rows_per_tile
        sis = (si0, si1)
        sgs = (sg0, sg1)
        sos = (so0, so1)

        def fire_idx_load(i, b):
            pltpu.async_copy(idx_hbm.at[pl.ds(base + i * XR, XR)],
                             idx_v.at[b], sis[b])

        def repack(b):
            @pl.loop(0, PR)
            def _(j):
                for k in range(PACK):
                    for q in range(D // 16):
                        packed_v[b, j, pl.ds(D * k + 16 * q, 16)] = (
                            rows_v[b, 4 * j + k, pl.ds(16 * q, 16)])

        for b in range(2):
            fire_idx_load(b, b)

        @pl.loop(0, n_steps, step=2)
        def _(g):
            copies = [[], []]
            for b in range(2):
                i = g + b
                pltpu.make_async_copy(
                    idx_hbm.at[pl.ds(0, XR)], idx_v.at[b], sis[b]).wait()
                for r in range(XR):
                    for c0, cw in CHUNKS:
                        copies[b].append(pltpu.async_copy(
                            table_hbm.at[idx_v.at[b, r, pl.ds(c0, cw)]],
                            rows_v.at[b, pl.ds(r * L + c0, cw)],
                            sgs[b]))
            for b in range(2):
                i = g + b
                for c in copies[b]:
                    c.wait()
                @pl.when(i >= 2)
                def _():
                    pltpu.make_async_copy(
                        packed_v.at[b], out_hbm.at[pl.ds(0, PR)],
                        sos[b]).wait()
                repack(b)
                pltpu.async_copy(
                    packed_v.at[b],
                    out_hbm.at[pl.ds((base + i * XR) * (L // PACK), PR)],
                    sos[b])
                @pl.when(i + 2 < n_steps)
                def _():
                    fire_idx_load(i + 2, b)

        for b in range(2):
            pltpu.make_async_copy(
                packed_v.at[b], out_hbm.at[pl.ds(0, PR)], sos[b]).wait()

    return gather_kernel(table, x_chunk)


MLP_BB = 32


def _mlp_body(emb_ref, wp_ref, wt0_ref, bt0_ref, wg0_ref, bg0_ref,
              wt1_ref, bt1_ref, wg1_ref, bg1_ref, out_ref):
    e = emb_ref[...].astype(jnp.bfloat16)
    h = jnp.dot(e, wp_ref[...], preferred_element_type=jnp.float32)
    for wt, bt, wg, bg in ((wt0_ref, bt0_ref, wg0_ref, bg0_ref),
                           (wt1_ref, bt1_ref, wg1_ref, bg1_ref)):
        hb = h.astype(jnp.bfloat16)
        g = jax.nn.sigmoid(
            jnp.dot(hb, wg[...], preferred_element_type=jnp.float32) + bg[...])
        t = jnp.maximum(
            jnp.dot(hb, wt[...], preferred_element_type=jnp.float32) + bt[...],
            0.0)
        h = g * t + (1.0 - g) * h
    out_ref[...] = h


def _tc_mlp(emb2, *weights):
    full = lambda shape: pl.BlockSpec(shape, lambda i: (0, 0))
    rows = emb2.shape[0]
    return pl.pallas_call(
        _mlp_body,
        grid=(rows // MLP_BLOCK,),
        in_specs=[
            pl.BlockSpec((MLP_BLOCK, DP), lambda i: (i, 0)),
            full((DP, HP)),
            full((HP, HP)), full((1, HP)),
            full((HP, HP)), full((1, HP)),
            full((HP, HP)), full((1, HP)),
            full((HP, HP)), full((1, HP)),
        ],
        out_specs=pl.BlockSpec((MLP_BLOCK, HP), lambda i: (i, 0)),
        out_shape=jax.ShapeDtypeStruct((rows, HP), jnp.float32),
        compiler_params=pltpu.CompilerParams(
            dimension_semantics=("parallel",),
        ),
    )(emb2, *weights)


def _block_diag4(w):
    a, b = w.shape
    out = jnp.zeros((PACK * a, PACK * b), w.dtype)
    for i in range(PACK):
        out = out.at[i * a:(i + 1) * a, i * b:(i + 1) * b].set(w)
    return out.astype(jnp.bfloat16)


N_CHUNKS = 4


def kernel(x, table, Wp, Wt0, bt0, Wg0, bg0, Wt1, bt1, Wg1, bg1):
    Wp2 = _block_diag4(Wp)
    args = [Wp2]
    for wt, bt, wg, bg in ((Wt0, bt0, Wg0, bg0), (Wt1, bt1, Wg1, bg1)):
        args += [_block_diag4(wt), jnp.tile(bt, PACK).reshape(1, HP),
                 _block_diag4(wg), jnp.tile(bg, PACK).reshape(1, HP)]

    BC = B // N_CHUNKS
    outs = []
    for c in range(N_CHUNKS):
        emb2 = _sc_gather(table, jax.lax.slice_in_dim(x, c * BC, (c + 1) * BC))
        h2 = _tc_mlp(emb2, *args)
        outs.append(h2.reshape(BC, L, H))
    return jnp.concatenate(outs, axis=0)


# speedup vs baseline: 1.2347x; 1.2347x over previous
import dataclasses
import functools

import jax
import jax.numpy as jnp
from jax.experimental import pallas as pl
from jax.experimental.pallas import tpu as pltpu
from jax.experimental.pallas import tpu_sc as plsc

V, D, H = 1000000, 32, 64
B, L = 4096, 200
N = B * L
PACK = 4
NP = N // PACK
DP, HP = D * PACK, H * PACK

GATHER_WINDOW = 128
MLP_BLOCK = 1024


def _sc_gather(table, x_chunk):
    mesh = plsc.VectorSubcoreMesh(core_axis_name="c", subcore_axis_name="s")
    BC = x_chunk.shape[0]

    n_tiles = 32
    rows_per_tile = BC // n_tiles
    XR = 4
    n_steps = rows_per_tile // XR
    assert n_steps % 2 == 0
    TOK = XR * L
    PR = TOK // PACK
    NPC = BC * L // PACK
    CHUNKS = ((0, 104), (104, 96))

    @functools.partial(
        pl.kernel,
        out_type=jax.ShapeDtypeStruct((NPC, DP), jnp.float32),
        mesh=mesh,
        scratch_types=[
            pltpu.VMEM((2, XR, L), jnp.int32),
            pltpu.VMEM((2, TOK, D), jnp.float32),
            pltpu.VMEM((2, PR, DP), jnp.float32),
            pltpu.SemaphoreType.DMA, pltpu.SemaphoreType.DMA,
            pltpu.SemaphoreType.DMA, pltpu.SemaphoreType.DMA,
            pltpu.SemaphoreType.DMA, pltpu.SemaphoreType.DMA,
        ],
        compiler_params=pltpu.CompilerParams(use_tc_tiling_on_sc=False),
    )
    def gather_kernel(table_hbm, idx_hbm, out_hbm, idx_v, rows_v,
                      packed_v, si0, si1, sg0, sg1, so0, so1):
        from jax import lax
        wid = lax.axis_index("s") * 2 + lax.axis_index("c")
        base = wid * rows_per_tile
        sis = (si0, si1)
        sgs = (sg0, sg1)
        sos = (so0, so1)

        def fire_idx_load(i, b):
            pltpu.async_copy(idx_hbm.at[pl.ds(base + i * XR, XR)],
                             idx_v.at[b], sis[b])

        def repack(b):
            @pl.loop(0, PR)
            def _(j):
                for k in range(PACK):
                    for q in range(D // 16):
                        packed_v[b, j, pl.ds(D * k + 16 * q, 16)] = (
                            rows_v[b, 4 * j + k, pl.ds(16 * q, 16)])

        for b in range(2):
            fire_idx_load(b, b)

        @pl.loop(0, n_steps, step=2)
        def _(g):
            copies = [[], []]
            for b in range(2):
                i = g + b
                pltpu.make_async_copy(
                    idx_hbm.at[pl.ds(0, XR)], idx_v.at[b], sis[b]).wait()
                for r in range(XR):
                    for c0, cw in CHUNKS:
                        copies[b].append(pltpu.async_copy(
                            table_hbm.at[idx_v.at[b, r, pl.ds(c0, cw)]],
                            rows_v.at[b, pl.ds(r * L + c0, cw)],
                            sgs[b]))
            for b in range(2):
                i = g + b
                for c in copies[b]:
                    c.wait()
                @pl.when(i >= 2)
                def _():
                    pltpu.make_async_copy(
                        packed_v.at[b], out_hbm.at[pl.ds(0, PR)],
                        sos[b]).wait()
                repack(b)
                pltpu.async_copy(
                    packed_v.at[b],
                    out_hbm.at[pl.ds((base + i * XR) * (L // PACK), PR)],
                    sos[b])
                @pl.when(i + 2 < n_steps)
                def _():
                    fire_idx_load(i + 2, b)

        for b in range(2):
            pltpu.make_async_copy(
                packed_v.at[b], out_hbm.at[pl.ds(0, PR)], sos[b]).wait()

    return gather_kernel(table, x_chunk)


MLP_BB = 32


def _mlp_body(emb_ref, wp_ref, wt0_ref, bt0_ref, wg0_ref, bg0_ref,
              wt1_ref, bt1_ref, wg1_ref, bg1_ref, out_ref):
    e = emb_ref[...].astype(jnp.bfloat16)
    h = jnp.dot(e, wp_ref[...], preferred_element_type=jnp.float32)
    for wt, bt, wg, bg in ((wt0_ref, bt0_ref, wg0_ref, bg0_ref),
                           (wt1_ref, bt1_ref, wg1_ref, bg1_ref)):
        hb = h.astype(jnp.bfloat16)
        g = jax.nn.sigmoid(
            jnp.dot(hb, wg[...], preferred_element_type=jnp.float32) + bg[...])
        t = jnp.maximum(
            jnp.dot(hb, wt[...], preferred_element_type=jnp.float32) + bt[...],
            0.0)
        h = g * t + (1.0 - g) * h
    out_ref[...] = h


def _tc_mlp(emb2, *weights):
    full = lambda shape: pl.BlockSpec(shape, lambda i: (0, 0))
    rows = emb2.shape[0]
    return pl.pallas_call(
        _mlp_body,
        grid=(rows // MLP_BLOCK,),
        in_specs=[
            pl.BlockSpec((MLP_BLOCK, DP), lambda i: (i, 0)),
            full((DP, HP)),
            full((HP, HP)), full((1, HP)),
            full((HP, HP)), full((1, HP)),
            full((HP, HP)), full((1, HP)),
            full((HP, HP)), full((1, HP)),
        ],
        out_specs=pl.BlockSpec((MLP_BLOCK, HP), lambda i: (i, 0)),
        out_shape=jax.ShapeDtypeStruct((rows, HP), jnp.float32),
        compiler_params=pltpu.CompilerParams(
            dimension_semantics=("parallel",),
        ),
    )(emb2, *weights)


def _block_diag4(w):
    a, b = w.shape
    out = jnp.zeros((PACK * a, PACK * b), w.dtype)
    for i in range(PACK):
        out = out.at[i * a:(i + 1) * a, i * b:(i + 1) * b].set(w)
    return out.astype(jnp.bfloat16)


def kernel(x, table, Wp, Wt0, bt0, Wg0, bg0, Wt1, bt1, Wg1, bg1):
    Wp2 = _block_diag4(Wp)
    args = [Wp2]
    for wt, bt, wg, bg in ((Wt0, bt0, Wg0, bg0), (Wt1, bt1, Wg1, bg1)):
        args += [_block_diag4(wt), jnp.tile(bt, PACK).reshape(1, HP),
                 _block_diag4(wg), jnp.tile(bg, PACK).reshape(1, HP)]

    emb2 = _sc_gather(table, x)
    h2 = _tc_mlp(emb2, *args)
    return h2.reshape(B, L, H)


# speedup vs baseline: 1.4170x; 1.1477x over previous
import dataclasses
import functools

import jax
import jax.numpy as jnp
from jax.experimental import pallas as pl
from jax.experimental.pallas import tpu as pltpu
from jax.experimental.pallas import tpu_sc as plsc

V, D, H = 1000000, 32, 64
B, L = 4096, 200
N = B * L
PACK = 4
NP = N // PACK
DP, HP = D * PACK, H * PACK

GATHER_WINDOW = 128
MLP_BLOCK = 1024


def _sc_gather(table, x_chunk):
    mesh = plsc.VectorSubcoreMesh(core_axis_name="c", subcore_axis_name="s")
    BC = x_chunk.shape[0]

    n_tiles = 32
    rows_per_tile = BC // n_tiles
    XR = 4
    n_steps = rows_per_tile // XR
    assert n_steps % 2 == 0
    TOK = XR * L
    PR = TOK // PACK
    NPC = BC * L // PACK
    CHUNKS = ((0, 104), (104, 96))

    @functools.partial(
        pl.kernel,
        out_type=jax.ShapeDtypeStruct((NPC, DP), jnp.float32),
        mesh=mesh,
        scratch_types=[
            pltpu.VMEM((2, XR, L), jnp.int32),
            pltpu.VMEM((2, TOK, D), jnp.float32),
            pltpu.VMEM((2, PR, DP), jnp.float32),
            pltpu.SemaphoreType.DMA, pltpu.SemaphoreType.DMA,
            pltpu.SemaphoreType.DMA, pltpu.SemaphoreType.DMA,
            pltpu.SemaphoreType.DMA, pltpu.SemaphoreType.DMA,
        ],
        compiler_params=pltpu.CompilerParams(use_tc_tiling_on_sc=False),
    )
    def gather_kernel(table_hbm, idx_hbm, out_hbm, idx_v, rows_v,
                      packed_v, si0, si1, sg0, sg1, so0, so1):
        from jax import lax
        wid = lax.axis_index("s") * 2 + lax.axis_index("c")
        base = wid * rows_per_tile
        sis = (si0, si1)
        sgs = (sg0, sg1)
        sos = (so0, so1)

        def fire_idx_load(i, b):
            pltpu.async_copy(idx_hbm.at[pl.ds(base + i * XR, XR)],
                             idx_v.at[b], sis[b])

        def repack(b):
            @pl.loop(0, PR)
            def _(j):
                for k in range(PACK):
                    for q in range(D // 16):
                        packed_v[b, j, pl.ds(D * k + 16 * q, 16)] = (
                            rows_v[b, 4 * j + k, pl.ds(16 * q, 16)])

        for b in range(2):
            fire_idx_load(b, b)

        @pl.loop(0, n_steps, step=2)
        def _(g):
            copies = [[], []]
            for b in range(2):
                i = g + b
                pltpu.make_async_copy(
                    idx_hbm.at[pl.ds(0, XR)], idx_v.at[b], sis[b]).wait()
                for r in range(XR):
                    for c0, cw in CHUNKS:
                        copies[b].append(pltpu.async_copy(
                            table_hbm.at[idx_v.at[b, r, pl.ds(c0, cw)]],
                            rows_v.at[b, pl.ds(r * L + c0, cw)],
                            sgs[b]))
            for b in range(2):
                i = g + b
                for c in copies[b]:
                    c.wait()
                @pl.when(i >= 2)
                def _():
                    pltpu.make_async_copy(
                        packed_v.at[b], out_hbm.at[pl.ds(0, PR)],
                        sos[b]).wait()
                repack(b)
                pltpu.async_copy(
                    packed_v.at[b],
                    out_hbm.at[pl.ds((base + i * XR) * (L // PACK), PR)],
                    sos[b])
                @pl.when(i + 2 < n_steps)
                def _():
                    fire_idx_load(i + 2, b)

        for b in range(2):
            pltpu.make_async_copy(
                packed_v.at[b], out_hbm.at[pl.ds(0, PR)], sos[b]).wait()

    return gather_kernel(table, x_chunk)


MLP_BB = 32


def _mlp_body(emb_ref, wp_ref, wt0_ref, bt0_ref, wg0_ref, bg0_ref,
              wt1_ref, bt1_ref, wg1_ref, bg1_ref, out_ref):
    e = emb_ref[...].astype(jnp.bfloat16)
    h = jnp.dot(e, wp_ref[...], preferred_element_type=jnp.float32)
    for wt, bt, wg, bg in ((wt0_ref, bt0_ref, wg0_ref, bg0_ref),
                           (wt1_ref, bt1_ref, wg1_ref, bg1_ref)):
        hb = h.astype(jnp.bfloat16)
        g = jax.nn.sigmoid(
            jnp.dot(hb, wg[...], preferred_element_type=jnp.float32) + bg[...])
        t = jnp.maximum(
            jnp.dot(hb, wt[...], preferred_element_type=jnp.float32) + bt[...],
            0.0)
        h = g * (t - h) + h
    bb = out_ref.shape[0]
    for k in range(PACK):
        v = h[:, k * H:(k + 1) * H].reshape(bb, L // PACK, H)
        out_ref[:, pl.Slice(k, L // PACK, PACK), :] = v


MLP_BB = 32


def _tc_mlp(emb2, *weights):
    full = lambda shape: pl.BlockSpec(shape, lambda i: (0, 0))
    blk_rows = MLP_BB * L // PACK
    return pl.pallas_call(
        _mlp_body,
        grid=(B // MLP_BB,),
        in_specs=[
            pl.BlockSpec((blk_rows, DP), lambda i: (i, 0)),
            full((DP, HP)),
            full((HP, HP)), full((1, HP)),
            full((HP, HP)), full((1, HP)),
            full((HP, HP)), full((1, HP)),
            full((HP, HP)), full((1, HP)),
        ],
        out_specs=pl.BlockSpec((MLP_BB, L, H), lambda i: (i, 0, 0)),
        out_shape=jax.ShapeDtypeStruct((B, L, H), jnp.float32),
        compiler_params=pltpu.CompilerParams(
            dimension_semantics=("parallel",),
        ),
    )(emb2, *weights)


def _block_diag4(w):
    a, b = w.shape
    out = jnp.zeros((PACK * a, PACK * b), w.dtype)
    for i in range(PACK):
        out = out.at[i * a:(i + 1) * a, i * b:(i + 1) * b].set(w)
    return out.astype(jnp.bfloat16)


def kernel(x, table, Wp, Wt0, bt0, Wg0, bg0, Wt1, bt1, Wg1, bg1):
    Wp2 = _block_diag4(Wp)
    args = [Wp2]
    for wt, bt, wg, bg in ((Wt0, bt0, Wg0, bg0), (Wt1, bt1, Wg1, bg1)):
        args += [_block_diag4(wt), jnp.tile(bt, PACK).reshape(1, HP),
                 _block_diag4(wg), jnp.tile(bg, PACK).reshape(1, HP)]

    emb2 = _sc_gather(table, x)
    return _tc_mlp(emb2, *args)


# speedup vs baseline: 1.4176x; 1.0004x over previous
import dataclasses
import functools

import jax
import jax.numpy as jnp
from jax.experimental import pallas as pl
from jax.experimental.pallas import tpu as pltpu
from jax.experimental.pallas import tpu_sc as plsc

V, D, H = 1000000, 32, 64
B, L = 4096, 200
N = B * L
PACK = 4
NP = N // PACK
DP, HP = D * PACK, H * PACK

GATHER_WINDOW = 128
MLP_BLOCK = 1024


def _sc_gather(table, x_chunk):
    mesh = plsc.VectorSubcoreMesh(core_axis_name="c", subcore_axis_name="s")
    BC = x_chunk.shape[0]

    n_tiles = 32
    rows_per_tile = BC // n_tiles
    XR = 4
    n_steps = rows_per_tile // XR
    assert n_steps % 2 == 0
    TOK = XR * L
    PR = TOK // PACK
    NPC = BC * L // PACK
    CHUNKS = ((0, 104), (104, 96))

    @functools.partial(
        pl.kernel,
        out_type=jax.ShapeDtypeStruct((NPC * DP,), jnp.float32),
        mesh=mesh,
        scratch_types=[
            pltpu.VMEM((2, XR, L), jnp.int32),
            pltpu.VMEM((2, TOK, D), jnp.float32),
            pltpu.VMEM((2, PR * DP), jnp.float32),
            pltpu.SemaphoreType.DMA, pltpu.SemaphoreType.DMA,
            pltpu.SemaphoreType.DMA, pltpu.SemaphoreType.DMA,
            pltpu.SemaphoreType.DMA, pltpu.SemaphoreType.DMA,
        ],
        compiler_params=pltpu.CompilerParams(use_tc_tiling_on_sc=False),
    )
    def gather_kernel(table_hbm, idx_hbm, out_hbm, idx_v, rows_v,
                      packed_v, si0, si1, sg0, sg1, so0, so1):
        from jax import lax
        wid = lax.axis_index("s") * 2 + lax.axis_index("c")
        base = wid * rows_per_tile
        sis = (si0, si1)
        sgs = (sg0, sg1)
        sos = (so0, so1)

        def fire_idx_load(i, b):
            pltpu.async_copy(idx_hbm.at[pl.ds(base + i * XR, XR)],
                             idx_v.at[b], sis[b])

        def repack(b):
            @pl.loop(0, PR)
            def _(j):
                for k in range(PACK):
                    for q in range(D // 16):
                        packed_v[b, pl.ds(DP * j + D * k + 16 * q, 16)] = (
                            rows_v[b, 4 * j + k, pl.ds(16 * q, 16)])

        for b in range(2):
            fire_idx_load(b, b)

        @pl.loop(0, n_steps, step=2)
        def _(g):
            copies = [[], []]
            for b in range(2):
                i = g + b
                pltpu.make_async_copy(
                    idx_hbm.at[pl.ds(0, XR)], idx_v.at[b], sis[b]).wait()
                for r in range(XR):
                    for c0, cw in CHUNKS:
                        copies[b].append(pltpu.async_copy(
                            table_hbm.at[idx_v.at[b, r, pl.ds(c0, cw)]],
                            rows_v.at[b, pl.ds(r * L + c0, cw)],
                            sgs[b]))
            for b in range(2):
                i = g + b
                for c in copies[b]:
                    c.wait()
                @pl.when(i >= 2)
                def _():
                    pltpu.make_async_copy(
                        packed_v.at[b], out_hbm.at[pl.ds(0, PR * DP)],
                        sos[b]).wait()
                repack(b)
                pltpu.async_copy(
                    packed_v.at[b],
                    out_hbm.at[pl.ds((base + i * XR) * L * D, PR * DP)],
                    sos[b])
                @pl.when(i + 2 < n_steps)
                def _():
                    fire_idx_load(i + 2, b)

        for b in range(2):
            pltpu.make_async_copy(
                packed_v.at[b], out_hbm.at[pl.ds(0, PR * DP)], sos[b]).wait()

    return gather_kernel(table, x_chunk)


MLP_BB = 32


def _mlp_body(emb_ref, wp_ref, wt0_ref, bt0_ref, wg0_ref, bg0_ref,
              wt1_ref, bt1_ref, wg1_ref, bg1_ref, out_ref):
    e = emb_ref[...].astype(jnp.bfloat16)
    h = jnp.dot(e, wp_ref[...], preferred_element_type=jnp.float32)
    for wt, bt, wg, bg in ((wt0_ref, bt0_ref, wg0_ref, bg0_ref),
                           (wt1_ref, bt1_ref, wg1_ref, bg1_ref)):
        hb = h.astype(jnp.bfloat16)
        g = jax.nn.sigmoid(
            jnp.dot(hb, wg[...], preferred_element_type=jnp.float32) + bg[...])
        t = jnp.maximum(
            jnp.dot(hb, wt[...], preferred_element_type=jnp.float32) + bt[...],
            0.0)
        h = g * (t - h) + h
    bb = out_ref.shape[0]
    for k in range(PACK):
        v = h[:, k * H:(k + 1) * H].reshape(bb, L // PACK, H)
        out_ref[:, pl.Slice(k, L // PACK, PACK), :] = v


MLP_BB = 32


def _tc_mlp(emb2, *weights):
    full = lambda shape: pl.BlockSpec(shape, lambda i: (0, 0))
    blk_rows = MLP_BB * L // PACK
    return pl.pallas_call(
        _mlp_body,
        grid=(B // MLP_BB,),
        in_specs=[
            pl.BlockSpec((blk_rows, DP), lambda i: (i, 0)),
            full((DP, HP)),
            full((HP, HP)), full((1, HP)),
            full((HP, HP)), full((1, HP)),
            full((HP, HP)), full((1, HP)),
            full((HP, HP)), full((1, HP)),
        ],
        out_specs=pl.BlockSpec((MLP_BB, L, H), lambda i: (i, 0, 0)),
        out_shape=jax.ShapeDtypeStruct((B, L, H), jnp.float32),
        compiler_params=pltpu.CompilerParams(
            dimension_semantics=("parallel",),
        ),
    )(emb2, *weights)


def _block_diag4(w):
    a, b = w.shape
    out = jnp.zeros((PACK * a, PACK * b), w.dtype)
    for i in range(PACK):
        out = out.at[i * a:(i + 1) * a, i * b:(i + 1) * b].set(w)
    return out.astype(jnp.bfloat16)


def kernel(x, table, Wp, Wt0, bt0, Wg0, bg0, Wt1, bt1, Wg1, bg1):
    Wp2 = _block_diag4(Wp)
    args = [Wp2]
    for wt, bt, wg, bg in ((Wt0, bt0, Wg0, bg0), (Wt1, bt1, Wg1, bg1)):
        args += [_block_diag4(wt), jnp.tile(bt, PACK).reshape(1, HP),
                 _block_diag4(wg), jnp.tile(bg, PACK).reshape(1, HP)]

    emb2 = _sc_gather(table, x).reshape(NP, DP)
    return _tc_mlp(emb2, *args)
